# baseline (speedup 1.0000x reference)
import functools

import jax
import jax.numpy as jnp
import numpy as np
from jax import lax
from jax.experimental import pallas as pl
from jax.experimental.pallas import tpu as pltpu
from jax.experimental.pallas import tpu_sc as plsc

D = 128
NC = 2
NS = 16
NW = NC * NS
EC = 64
PC = 80

_MESH = plsc.VectorSubcoreMesh(
    core_axis_name="c", subcore_axis_name="s", num_cores=NC, num_subcores=NS)


def _pre_body(x_ref, wp_ref, bp_ref, q0_ref, bq0_ref, h_ref, n0_ref):
    h = jnp.dot(x_ref[...], wp_ref[...], preferred_element_type=jnp.float32)
    h = h + bp_ref[...]
    h_ref[...] = h
    n0 = jnp.dot(h, q0_ref[...], preferred_element_type=jnp.float32)
    n0_ref[...] = jnp.maximum(n0 + bq0_ref[...], 0.0).astype(jnp.bfloat16)


def _pre(x, W_proj, b_proj, Q0, bQ0, NP, B):
    grid = NP // B
    row = pl.BlockSpec((B, D), lambda i: (i, 0))
    full = pl.BlockSpec((D, D), lambda i: (0, 0))
    vec = pl.BlockSpec((1, D), lambda i: (0, 0))
    return pl.pallas_call(
        _pre_body,
        grid=(grid,),
        in_specs=[row, full, vec, full, vec],
        out_specs=[row, row],
        out_shape=[jax.ShapeDtypeStruct((NP, D), jnp.float32),
                   jax.ShapeDtypeStruct((NP, D), jnp.bfloat16)],
    )(x, W_proj, b_proj.reshape(1, D), Q0, bQ0.reshape(1, D))


def _norm_rows(z):
    nrm = jnp.sqrt(jnp.sum(z * z, axis=1, keepdims=True))
    nrm = jnp.where(nrm == 0.0, 1.0, nrm)
    return z / nrm


def _combine0_body(aggp_ref, wsp_ref, h_ref, wt_ref, wb_ref, bw_ref,
                   q1_ref, bq1_ref, h1_ref, n1_ref):
    agg = aggp_ref[0] + aggp_ref[1]
    ws = jnp.maximum(wsp_ref[0] + wsp_ref[1], 1.0)
    a = agg / ws[:, None]
    z = (jnp.dot(a, wt_ref[...], preferred_element_type=jnp.float32)
         + jnp.dot(h_ref[...], wb_ref[...], preferred_element_type=jnp.float32)
         + bw_ref[...])
    h1 = _norm_rows(jnp.maximum(z, 0.0))
    h1_ref[...] = h1
    n1 = jnp.dot(h1, q1_ref[...], preferred_element_type=jnp.float32)
    n1_ref[...] = jnp.maximum(n1 + bq1_ref[...], 0.0).astype(jnp.bfloat16)


def _combine0(aggp, wsp, h, Wt, Wb, bW, Q1, bQ1, NP, B):
    grid = NP // B
    row = pl.BlockSpec((B, D), lambda i: (i, 0))
    full = pl.BlockSpec((D, D), lambda i: (0, 0))
    vec = pl.BlockSpec((1, D), lambda i: (0, 0))
    a3 = pl.BlockSpec((NC, B, D), lambda i: (0, i, 0))
    a2 = pl.BlockSpec((NC, B), lambda i: (0, i))
    return pl.pallas_call(
        _combine0_body,
        grid=(grid,),
        in_specs=[a3, a2, row, full, full, vec, full, vec],
        out_specs=[row, row],
        out_shape=[jax.ShapeDtypeStruct((NP, D), jnp.float32),
                   jax.ShapeDtypeStruct((NP, D), jnp.bfloat16)],
    )(aggp, wsp, h, Wt, Wb, bW.reshape(1, D), Q1, bQ1.reshape(1, D))


def _combine1_body(aggp_ref, wsp_ref, h1_ref, h_ref, wt_ref, wb_ref, bw_ref,
                   out_ref):
    agg = aggp_ref[0] + aggp_ref[1]
    ws = jnp.maximum(wsp_ref[0] + wsp_ref[1], 1.0)
    a = agg / ws[:, None]
    z = (jnp.dot(a, wt_ref[...], preferred_element_type=jnp.float32)
         + jnp.dot(h1_ref[...], wb_ref[...], preferred_element_type=jnp.float32)
         + bw_ref[...])
    out_ref[...] = h_ref[...] + _norm_rows(jnp.maximum(z, 0.0))


def _combine1(aggp, wsp, h1, h, Wt, Wb, bW, NP, B):
    grid = NP // B
    row = pl.BlockSpec((B, D), lambda i: (i, 0))
    full = pl.BlockSpec((D, D), lambda i: (0, 0))
    vec = pl.BlockSpec((1, D), lambda i: (0, 0))
    a3 = pl.BlockSpec((NC, B, D), lambda i: (0, i, 0))
    a2 = pl.BlockSpec((NC, B), lambda i: (0, i))
    return pl.pallas_call(
        _combine1_body,
        grid=(grid,),
        in_specs=[a3, a2, row, row, full, full, vec],
        out_specs=row,
        out_shape=jax.ShapeDtypeStruct((NP, D), jnp.float32),
    )(aggp, wsp, h1, h, Wt, Wb, bW.reshape(1, D))


NBUF = 3
NIB = 3


def _scale_chunk(bf_b, fout_b, wgrp, pr, b):
    prv = jnp.full((16,), pr, jnp.int32)
    bv = jnp.full((16,), b, jnp.int32)

    def edge(j, _):
        wvec = plsc.load_gather(wgrp, [prv, bv, jnp.full((16,), j, jnp.int32)])
        for k in range(D // 32):
            v = bf_b[j, pl.ds(k * 16, 16)]
            lo = plsc.bitcast(jnp.left_shift(v, 16), jnp.float32)
            hi = plsc.bitcast(jnp.bitwise_and(v, jnp.int32(-65536)),
                              jnp.float32)
            fout_b[j, pl.ds(k * 32, 16)] = lo * wvec
            fout_b[j, pl.ds(k * 32 + 16, 16)] = hi * wvec
        return 0
    lax.fori_loop(0, EC, edge, 0, unroll=2)


def _feat_perm():
    blocks = [np.concatenate([np.arange(0, 32, 2), np.arange(1, 32, 2)]) + 32 * k
              for k in range(D // 32)]
    return np.concatenate(blocks)


def _seg_body(with_ws, n_chunks, npad, n_hbm, comb_hbm, w_hbm, *rest):
    rest = list(rest)
    agg_out = rest.pop(0)
    ws_out = rest.pop(0) if with_ws else None
    comb = rest.pop(0)
    wgrp = rest.pop(0)
    bf = [rest.pop(0) for _ in range(NBUF)]
    fout = [rest.pop(0) for _ in range(NBUF)]
    zws_v = rest.pop(0) if with_ws else None
    agg_sh = rest.pop(0)
    ws_sh = rest.pop(0) if with_ws else None
    g = [rest.pop(0) for _ in range(NBUF)]
    s = [rest.pop(0) for _ in range(NBUF)]
    isem = rest.pop(0)
    wsem = rest.pop(0) if with_ws else None

    cid = lax.axis_index("c")
    sid = lax.axis_index("s")
    wid = sid * NC + cid
    rpt = npad // NS
    base = sid * rpt
    cbase = wid * n_chunks
    groups = n_chunks // NBUF

    def zrow(i, _):
        for k in range(D // 16):
            fout[0][i, pl.ds(k * 16, 16)] = jnp.zeros((16,), jnp.float32)
        return 0
    lax.fori_loop(0, EC, zrow, 0)
    for r in range(rpt // EC):
        pltpu.sync_copy(fout[0], agg_sh.at[pl.ds(base + r * EC, EC)])
    if with_ws:
        def zws(i, _):
            zws_v[pl.ds(i * 16, 16)] = jnp.zeros((16,), jnp.float32)
            return 0
        lax.fori_loop(0, rpt // 16, zws, 0)
        pltpu.sync_copy(zws_v, ws_sh.at[pl.ds(base, rpt)])

    def _iload(grp, buf, sem_start):
        hslice = comb_hbm.at[pl.ds(cbase + grp * NBUF, NBUF)]
        wslice = w_hbm.at[pl.ds(cbase + grp * NBUF, NBUF)]
        if sem_start:
            pltpu.async_copy(hslice, comb.at[buf], isem)
            pltpu.async_copy(wslice, wgrp.at[buf], isem)
        else:
            pltpu.make_async_copy(hslice, comb.at[buf], isem).wait()
            pltpu.make_async_copy(wslice, wgrp.at[buf], isem).wait()

    pltpu.sync_copy(comb_hbm.at[pl.ds(cbase, NBUF)], comb.at[0])
    pltpu.sync_copy(w_hbm.at[pl.ds(cbase, NBUF)], wgrp.at[0])
    _iload(1, 1, True)
    plsc.subcore_barrier()

    pltpu.async_copy(n_hbm.at[comb.at[0, 0, 0]], bf[0], g[0])
    pltpu.async_copy(n_hbm.at[comb.at[0, 1, 0]], bf[1], g[1])

    def group(p, _):
        pr = p % NIB
        prn = (p + 1) % NIB
        prp = (p - 1) % NIB
        for b in range(NBUF):
            i = p * NBUF + b
            if b == 1:
                pl.when(p < groups - 1)(lambda: _iload(p + 1, prn, False))
            pltpu.make_async_copy(n_hbm.at[comb.at[pr, b, 0]], bf[b],
                                  g[b]).wait()
            bm2 = (b + 2) % NBUF
            gpr = pr if b == 0 else prn
            gb = (b + 2) % NBUF

            def _gstart():
                pltpu.async_copy(n_hbm.at[comb.at[gpr, gb, 0]], bf[bm2],
                                 g[bm2])
            if b == 0:
                _gstart()
            else:
                pl.when(p < groups - 1)(_gstart)
            def _swait():
                pltpu.make_async_copy(fout[b],
                                      agg_sh.at[comb.at[prp, b, 1]],
                                      s[b]).wait()
            pl.when(p >= 1)(_swait)
            _scale_chunk(bf[b], fout[b], wgrp, pr, b)
            if with_ws:
                bm1 = (b - 1) % NBUF
                wpr = prp if b == 0 else pr

                def _wswait():
                    pltpu.make_async_copy(
                        wgrp.at[wpr, bm1], ws_sh.at[comb.at[wpr, bm1, 1]],
                        wsem).wait()
                if b == 0:
                    pl.when(p >= 1)(_wswait)
                else:
                    _wswait()
                pltpu.async_copy(wgrp.at[pr, b], ws_sh.at[comb.at[pr, b, 1]],
                                 wsem, add=True)
            pltpu.async_copy(fout[b], agg_sh.at[comb.at[pr, b, 1]], s[b],
                             add=True)
        pl.when(p < groups - 2)(lambda: _iload(p + 2, (p + 2) % NIB, True))
        return 0
    lax.fori_loop(0, groups, group, 0)

    lpr = (groups - 1) % NIB
    for b in range(NBUF):
        pltpu.make_async_copy(fout[b], agg_sh.at[comb.at[lpr, b, 1]],
                              s[b]).wait()
    if with_ws:
        pltpu.make_async_copy(wgrp.at[lpr, NBUF - 1],
                              ws_sh.at[comb.at[lpr, NBUF - 1, 1]],
                              wsem).wait()
    plsc.subcore_barrier()

    pltpu.sync_copy(agg_sh.at[pl.ds(base, rpt)],
                    agg_out.at[cid, pl.ds(base, rpt)])
    if with_ws:
        pltpu.sync_copy(ws_sh.at[pl.ds(base, rpt)],
                        ws_out.at[cid, pl.ds(base, rpt)])


def _segment_sum(n_tab, comb, w2, npad, with_ws):
    n_chunks = comb.shape[0] // NW
    out_type = [jax.ShapeDtypeStruct((NC, npad, D), jnp.float32)]
    scratch = [pltpu.VMEM((NIB, NBUF, 2, EC), jnp.int32),
               pltpu.VMEM((NIB, NBUF, EC), jnp.float32)]
    scratch += [pltpu.VMEM((EC, D // 2), jnp.int32)] * NBUF
    scratch += [pltpu.VMEM((EC, D), jnp.float32)] * NBUF
    if with_ws:
        out_type.append(jax.ShapeDtypeStruct((NC, npad), jnp.float32))
        scratch.append(pltpu.VMEM((npad // NS,), jnp.float32))
    scratch.append(pltpu.VMEM_SHARED((npad, D), jnp.float32))
    if with_ws:
        scratch.append(pltpu.VMEM_SHARED((npad,), jnp.float32))
    scratch += [pltpu.SemaphoreType.DMA] * (2 * NBUF + 1)
    if with_ws:
        scratch.append(pltpu.SemaphoreType.DMA)

    body = functools.partial(_seg_body, with_ws, n_chunks, npad)
    fn = pl.kernel(body, out_type=tuple(out_type), mesh=_MESH,
                   scratch_types=tuple(scratch),
                   compiler_params=pltpu.CompilerParams(
                       needs_layout_passes=False,
                       use_tc_tiling_on_sc=False))
    return fn(n_tab, comb, w2)


def _score_body(n_chunks, h_hbm, pu_hbm, pv_hbm, nu_hbm, nv_hbm, bias_hbm,
                out_hbm, *rest):
    rest = list(rest)
    idx = [rest.pop(0) for _ in range(4)]
    bv = [rest.pop(0) for _ in range(4)]
    rbufs = [[rest.pop(0) for _ in range(4)] for _ in range(2)]
    dots_v = rest.pop(0)
    out_v = rest.pop(0)
    g = [rest.pop(0) for _ in range(2)]
    bsem = rest.pop(0)

    cid = lax.axis_index("c")
    sid = lax.axis_index("s")
    wid = sid * NC + cid
    ppw = n_chunks * PC
    pbase = wid * ppw

    ih = [pu_hbm, pv_hbm, nu_hbm, nv_hbm]
    for t in range(4):
        pltpu.sync_copy(ih[t].at[pl.ds(pbase, ppw)], idx[t])

    def _rgather(c, q, sem_start):
        for t in range(4):
            d = pltpu.make_async_copy(
                h_hbm.at[idx[t].at[pl.ds(c * PC, PC)]], rbufs[q][t], g[q])
            if sem_start:
                d.start()
            else:
                d.wait()

    def _bgather(sem_start):
        for t in range(4):
            for c in range(n_chunks):
                d = pltpu.make_async_copy(
                    bias_hbm.at[idx[t].at[pl.ds(c * PC, PC)]],
                    bv[t].at[pl.ds(c * PC, PC)], bsem)
                if sem_start:
                    d.start()
                else:
                    d.wait()
    _bgather(True)
    _rgather(0, 0, True)
    _bgather(False)

    for c in range(n_chunks):
        q = c % 2
        _rgather(c, q, False)
        if c + 1 < n_chunks:
            _rgather(c + 1, 1 - q, True)
        ru, rv, su, sv = rbufs[q]

        def pair(j, _):
            acc = jnp.zeros((16,), jnp.float32)
            for k in range(D // 16):
                sl = pl.ds(k * 16, 16)
                acc = acc + su[j, sl] * sv[j, sl]
                acc = acc - ru[j, sl] * rv[j, sl]
            dots_v[j, :] = acc
            return 0
        lax.fori_loop(0, PC, pair, 0, unroll=2)

        lanes = lax.iota(jnp.int32, 16)
        for gg in range(PC // 16):
            sl = pl.ds(c * PC + gg * 16, 16)
            rows = lanes + gg * 16
            tot = jnp.full((16,), 1.0, jnp.float32)
            for l in range(16):
                tot = tot + plsc.load_gather(
                    dots_v, [rows, jnp.full((16,), l, jnp.int32)])
            bterm = bv[2][sl] + bv[3][sl] - bv[0][sl] - bv[1][sl]
            out_v[pl.ds(gg * 16, 16)] = jnp.maximum(tot + bterm, 0.0)
        pltpu.sync_copy(out_v, out_hbm.at[pl.ds(pbase + c * PC, PC)])


def _score(h_item, pu, pv, nu, nv, bias, npad):
    pp = pu.shape[0]
    n_chunks = pp // (NW * PC)
    ppw = n_chunks * PC
    scratch = [pltpu.VMEM((ppw,), jnp.int32)] * 4
    scratch += [pltpu.VMEM((ppw,), jnp.float32)] * 4
    scratch += [pltpu.VMEM((PC, D), jnp.float32)] * 8
    scratch += [pltpu.VMEM((PC, 16), jnp.float32),
                pltpu.VMEM((PC,), jnp.float32)]
    scratch += [pltpu.SemaphoreType.DMA] * 3
    body = functools.partial(_score_body, n_chunks)
    fn = pl.kernel(body, out_type=jax.ShapeDtypeStruct((pp,), jnp.float32),
                   mesh=_MESH, scratch_types=tuple(scratch),
                   compiler_params=pltpu.CompilerParams(
                       needs_layout_passes=False))
    return fn(h_item, pu, pv, nu, nv, bias)


def _round_up(a, b):
    return (a + b - 1) // b * b


def kernel(x, edge_index, edge_weights, pos_edge_index, neg_edge_index,
           W_proj, b_proj, Q0, bQ0, W0, bW0, Q1, bQ1, W1, bW1, scorer_bias):
    n = x.shape[0]
    e = edge_index.shape[1]
    p = pos_edge_index.shape[1]

    npad = _round_up(n, NS * EC)
    B = 512
    npad = _round_up(npad, B)

    ep = _round_up(e, NW * EC * NBUF)
    ppad = _round_up(p, NW * PC)

    xp = jnp.pad(x, ((0, npad - n), (0, 0)))
    biasp = jnp.pad(scorer_bias, (0, npad - n))

    fill = (jnp.arange(ep - e, dtype=jnp.int32) % n).astype(jnp.int32)
    src = jnp.concatenate([edge_index[0], fill]).reshape(ep // EC, 1, EC)
    dst = jnp.concatenate([edge_index[1], fill]).reshape(ep // EC, 1, EC)
    w2 = jnp.concatenate([edge_weights,
                          jnp.zeros((ep - e,), jnp.float32)]
                         ).reshape(ep // EC, EC)
    comb = jnp.concatenate([src, dst], axis=1)

    pfill = jnp.zeros((ppad - p,), jnp.int32)
    pu = jnp.concatenate([pos_edge_index[0], pfill])
    pv = jnp.concatenate([pos_edge_index[1], pfill])
    nu = jnp.concatenate([neg_edge_index[0], pfill])
    nv = jnp.concatenate([neg_edge_index[1], pfill])

    perm = _feat_perm()

    def _pack(nt):
        return lax.bitcast_convert_type(
            nt.reshape(npad, D // 2, 2), jnp.int32)

    h, n0 = _pre(xp, W_proj, b_proj, Q0, bQ0, npad, B)
    aggp0, wsp = _segment_sum(_pack(n0), comb, w2, npad, with_ws=True)
    h1, n1 = _combine0(aggp0, wsp, h, W0[perm], W0[D:], bW0, Q1, bQ1, npad, B)
    (aggp1,) = _segment_sum(_pack(n1), comb, w2, npad, with_ws=False)
    h_item = _combine1(aggp1, wsp, h1, h, W1[perm], W1[D:], bW1, npad, B)
    scores = _score(h_item, pu, pv, nu, nv, biasp, npad)
    return scores[:p]


# speedup vs baseline: 1.6151x; 1.6151x over previous
import functools

import jax
import jax.numpy as jnp
import numpy as np
from jax import lax
from jax.experimental import pallas as pl
from jax.experimental.pallas import tpu as pltpu
from jax.experimental.pallas import tpu_sc as plsc

D = 128
NC = 2
NS = 16
NW = NC * NS
EC = 64
PC = 80

_MESH = plsc.VectorSubcoreMesh(
    core_axis_name="c", subcore_axis_name="s", num_cores=NC, num_subcores=NS)


def _pre_body(x_ref, wp_ref, bp_ref, q0_ref, bq0_ref, h_ref, n0_ref):
    h = jnp.dot(x_ref[...], wp_ref[...], preferred_element_type=jnp.float32)
    h = h + bp_ref[...]
    h_ref[...] = h
    n0 = jnp.dot(h, q0_ref[...], preferred_element_type=jnp.float32)
    n0_ref[...] = jnp.maximum(n0 + bq0_ref[...], 0.0)


def _pre(x, W_proj, b_proj, Q0, bQ0, NP, B):
    grid = NP // B
    row = pl.BlockSpec((B, D), lambda i: (i, 0))
    full = pl.BlockSpec((D, D), lambda i: (0, 0))
    vec = pl.BlockSpec((1, D), lambda i: (0, 0))
    return pl.pallas_call(
        _pre_body,
        grid=(grid,),
        in_specs=[row, full, vec, full, vec],
        out_specs=[row, row],
        out_shape=[jax.ShapeDtypeStruct((NP, D), jnp.float32)] * 2,
    )(x, W_proj, b_proj.reshape(1, D), Q0, bQ0.reshape(1, D))


def _norm_rows(z):
    nrm = jnp.sqrt(jnp.sum(z * z, axis=1, keepdims=True))
    nrm = jnp.where(nrm == 0.0, 1.0, nrm)
    return z / nrm


def _combine0_body(aggp_ref, wsp_ref, h_ref, wt_ref, wb_ref, bw_ref,
                   q1_ref, bq1_ref, h1_ref, n1_ref):
    agg = aggp_ref[0] + aggp_ref[1]
    ws = jnp.maximum(wsp_ref[0] + wsp_ref[1], 1.0)
    a = agg / ws[:, None]
    z = (jnp.dot(a, wt_ref[...], preferred_element_type=jnp.float32)
         + jnp.dot(h_ref[...], wb_ref[...], preferred_element_type=jnp.float32)
         + bw_ref[...])
    h1 = _norm_rows(jnp.maximum(z, 0.0))
    h1_ref[...] = h1
    n1 = jnp.dot(h1, q1_ref[...], preferred_element_type=jnp.float32)
    n1_ref[...] = jnp.maximum(n1 + bq1_ref[...], 0.0)


def _combine0(aggp, wsp, h, Wt, Wb, bW, Q1, bQ1, NP, B):
    grid = NP // B
    row = pl.BlockSpec((B, D), lambda i: (i, 0))
    full = pl.BlockSpec((D, D), lambda i: (0, 0))
    vec = pl.BlockSpec((1, D), lambda i: (0, 0))
    a3 = pl.BlockSpec((NC, B, D), lambda i: (0, i, 0))
    a2 = pl.BlockSpec((NC, B), lambda i: (0, i))
    return pl.pallas_call(
        _combine0_body,
        grid=(grid,),
        in_specs=[a3, a2, row, full, full, vec, full, vec],
        out_specs=[row, row],
        out_shape=[jax.ShapeDtypeStruct((NP, D), jnp.float32)] * 2,
    )(aggp, wsp, h, Wt, Wb, bW.reshape(1, D), Q1, bQ1.reshape(1, D))


def _combine1_body(aggp_ref, wsp_ref, h1_ref, h_ref, wt_ref, wb_ref, bw_ref,
                   out_ref):
    agg = aggp_ref[0] + aggp_ref[1]
    ws = jnp.maximum(wsp_ref[0] + wsp_ref[1], 1.0)
    a = agg / ws[:, None]
    z = (jnp.dot(a, wt_ref[...], preferred_element_type=jnp.float32)
         + jnp.dot(h1_ref[...], wb_ref[...], preferred_element_type=jnp.float32)
         + bw_ref[...])
    out_ref[...] = h_ref[...] + _norm_rows(jnp.maximum(z, 0.0))


def _combine1(aggp, wsp, h1, h, Wt, Wb, bW, NP, B):
    grid = NP // B
    row = pl.BlockSpec((B, D), lambda i: (i, 0))
    full = pl.BlockSpec((D, D), lambda i: (0, 0))
    vec = pl.BlockSpec((1, D), lambda i: (0, 0))
    a3 = pl.BlockSpec((NC, B, D), lambda i: (0, i, 0))
    a2 = pl.BlockSpec((NC, B), lambda i: (0, i))
    return pl.pallas_call(
        _combine1_body,
        grid=(grid,),
        in_specs=[a3, a2, row, row, full, full, vec],
        out_specs=row,
        out_shape=jax.ShapeDtypeStruct((NP, D), jnp.float32),
    )(aggp, wsp, h1, h, Wt, Wb, bW.reshape(1, D))


NBUF = 4
NIB = 3


def _scale_chunk(rows_b, wgrp, pr, b):
    prv = jnp.full((16,), pr, jnp.int32)
    bv = jnp.full((16,), b, jnp.int32)

    def edge(j, _):
        wvec = plsc.load_gather(wgrp, [prv, bv, jnp.full((16,), j, jnp.int32)])
        for k in range(D // 16):
            sl = pl.ds(k * 16, 16)
            rows_b[j, sl] = rows_b[j, sl] * wvec
        return 0
    lax.fori_loop(0, EC, edge, 0, unroll=2)


def _seg_body(with_ws, n_chunks, npad, n_hbm, comb_hbm, w_hbm, *rest):
    rest = list(rest)
    agg_out = rest.pop(0)
    ws_out = rest.pop(0) if with_ws else None
    comb = rest.pop(0)
    wgrp = rest.pop(0)
    rows = [rest.pop(0) for _ in range(NBUF)]
    zws_v = rest.pop(0) if with_ws else None
    agg_sh = rest.pop(0)
    ws_sh = rest.pop(0) if with_ws else None
    g = [rest.pop(0) for _ in range(NBUF)]
    s = [rest.pop(0) for _ in range(NBUF)]
    isem = rest.pop(0)
    wsem = rest.pop(0) if with_ws else None

    cid = lax.axis_index("c")
    sid = lax.axis_index("s")
    wid = sid * NC + cid
    rpt = npad // NS
    base = sid * rpt
    cbase = wid * n_chunks
    groups = n_chunks // NBUF

    def zrow(i, _):
        for k in range(D // 16):
            rows[0][i, pl.ds(k * 16, 16)] = jnp.zeros((16,), jnp.float32)
        return 0
    lax.fori_loop(0, EC, zrow, 0)
    for r in range(rpt // EC):
        pltpu.sync_copy(rows[0], agg_sh.at[pl.ds(base + r * EC, EC)])
    if with_ws:
        def zws(i, _):
            zws_v[pl.ds(i * 16, 16)] = jnp.zeros((16,), jnp.float32)
            return 0
        lax.fori_loop(0, rpt // 16, zws, 0)
        pltpu.sync_copy(zws_v, ws_sh.at[pl.ds(base, rpt)])

    def _iload(grp, buf, sem_start):
        hslice = comb_hbm.at[pl.ds(cbase + grp * NBUF, NBUF)]
        wslice = w_hbm.at[pl.ds(cbase + grp * NBUF, NBUF)]
        if sem_start:
            pltpu.async_copy(hslice, comb.at[buf], isem)
            pltpu.async_copy(wslice, wgrp.at[buf], isem)
        else:
            pltpu.make_async_copy(hslice, comb.at[buf], isem).wait()
            pltpu.make_async_copy(wslice, wgrp.at[buf], isem).wait()

    pltpu.sync_copy(comb_hbm.at[pl.ds(cbase, NBUF)], comb.at[0])
    pltpu.sync_copy(w_hbm.at[pl.ds(cbase, NBUF)], wgrp.at[0])
    _iload(1, 1, True)
    plsc.subcore_barrier()

    pltpu.async_copy(n_hbm.at[comb.at[0, 0, 0]], rows[0], g[0])
    pltpu.async_copy(n_hbm.at[comb.at[0, 1, 0]], rows[1], g[1])

    def group(p, _):
        pr = p % NIB
        prn = (p + 1) % NIB
        prp = (p - 1) % NIB
        pl.when(p < groups - 1)(lambda: _iload(p + 1, prn, False))
        for b in range(NBUF):
            i = p * NBUF + b
            if b == 2:
                pl.when(p < groups - 2)(
                    lambda: _iload(p + 2, (p + 2) % NIB, True))
            pltpu.make_async_copy(n_hbm.at[comb.at[pr, b, 0]], rows[b],
                                  g[b]).wait()
            _scale_chunk(rows[b], wgrp, pr, b)
            if with_ws:
                bm1 = (b - 1) % NBUF
                wpr = prp if b == 0 else pr

                def _wswait():
                    pltpu.make_async_copy(
                        wgrp.at[wpr, bm1], ws_sh.at[comb.at[wpr, bm1, 1]],
                        wsem).wait()
                if b == 0:
                    pl.when(p >= 1)(_wswait)
                else:
                    _wswait()
                pltpu.async_copy(wgrp.at[pr, b], ws_sh.at[comb.at[pr, b, 1]],
                                 wsem, add=True)
            pltpu.async_copy(rows[b], agg_sh.at[comb.at[pr, b, 1]], s[b],
                             add=True)
            bm2 = (b + 2) % NBUF
            spr = prp if b < 2 else pr
            gpr = pr if b < 2 else prn
            gb = b + 2 if b < 2 else b - 2

            def _swait():
                pltpu.make_async_copy(rows[bm2],
                                      agg_sh.at[comb.at[spr, bm2, 1]],
                                      s[bm2]).wait()

            def _gstart():
                pltpu.async_copy(n_hbm.at[comb.at[gpr, gb, 0]], rows[bm2],
                                 g[bm2])

            if b < 2:
                pl.when(p >= 1)(_swait)
                _gstart()
            else:
                _swait()
                pl.when(p < groups - 1)(_gstart)
        return 0
    lax.fori_loop(0, groups, group, 0)

    lpr = (groups - 1) % NIB
    pltpu.make_async_copy(rows[2], agg_sh.at[comb.at[lpr, 2, 1]],
                          s[2]).wait()
    pltpu.make_async_copy(rows[3], agg_sh.at[comb.at[lpr, 3, 1]],
                          s[3]).wait()
    if with_ws:
        pltpu.make_async_copy(wgrp.at[lpr, NBUF - 1],
                              ws_sh.at[comb.at[lpr, NBUF - 1, 1]],
                              wsem).wait()
    plsc.subcore_barrier()

    pltpu.sync_copy(agg_sh.at[pl.ds(base, rpt)],
                    agg_out.at[cid, pl.ds(base, rpt)])
    if with_ws:
        pltpu.sync_copy(ws_sh.at[pl.ds(base, rpt)],
                        ws_out.at[cid, pl.ds(base, rpt)])


def _segment_sum(n_tab, comb, w2, npad, with_ws):
    n_chunks = comb.shape[0] // NW
    out_type = [jax.ShapeDtypeStruct((NC, npad, D), jnp.float32)]
    scratch = [pltpu.VMEM((NIB, NBUF, 2, EC), jnp.int32),
               pltpu.VMEM((NIB, NBUF, EC), jnp.float32)]
    scratch += [pltpu.VMEM((EC, D), jnp.float32)] * NBUF
    if with_ws:
        out_type.append(jax.ShapeDtypeStruct((NC, npad), jnp.float32))
        scratch.append(pltpu.VMEM((npad // NS,), jnp.float32))
    scratch.append(pltpu.VMEM_SHARED((npad, D), jnp.float32))
    if with_ws:
        scratch.append(pltpu.VMEM_SHARED((npad,), jnp.float32))
    scratch += [pltpu.SemaphoreType.DMA] * (2 * NBUF + 1)
    if with_ws:
        scratch.append(pltpu.SemaphoreType.DMA)

    body = functools.partial(_seg_body, with_ws, n_chunks, npad)
    fn = pl.kernel(body, out_type=tuple(out_type), mesh=_MESH,
                   scratch_types=tuple(scratch),
                   compiler_params=pltpu.CompilerParams(
                       needs_layout_passes=False))
    return fn(n_tab, comb, w2)


def _score_body(n_chunks, h_hbm, pu_hbm, pv_hbm, nu_hbm, nv_hbm, bias_hbm,
                out_hbm, *rest):
    rest = list(rest)
    idx = [rest.pop(0) for _ in range(4)]
    bv = [rest.pop(0) for _ in range(4)]
    rbufs = [[rest.pop(0) for _ in range(4)] for _ in range(2)]
    dots_v = rest.pop(0)
    out_v = rest.pop(0)
    g = [rest.pop(0) for _ in range(2)]
    bsem = rest.pop(0)

    cid = lax.axis_index("c")
    sid = lax.axis_index("s")
    wid = sid * NC + cid
    ppw = n_chunks * PC
    pbase = wid * ppw

    ih = [pu_hbm, pv_hbm, nu_hbm, nv_hbm]
    for t in range(4):
        pltpu.sync_copy(ih[t].at[pl.ds(pbase, ppw)], idx[t])

    def _rgather(c, q, sem_start):
        for t in range(4):
            d = pltpu.make_async_copy(
                h_hbm.at[idx[t].at[pl.ds(c * PC, PC)]], rbufs[q][t], g[q])
            if sem_start:
                d.start()
            else:
                d.wait()

    def _bgather(sem_start):
        for t in range(4):
            for c in range(n_chunks):
                d = pltpu.make_async_copy(
                    bias_hbm.at[idx[t].at[pl.ds(c * PC, PC)]],
                    bv[t].at[pl.ds(c * PC, PC)], bsem)
                if sem_start:
                    d.start()
                else:
                    d.wait()
    _bgather(True)
    _rgather(0, 0, True)
    _bgather(False)

    for c in range(n_chunks):
        q = c % 2
        _rgather(c, q, False)
        if c + 1 < n_chunks:
            _rgather(c + 1, 1 - q, True)
        ru, rv, su, sv = rbufs[q]

        def pair(j, _):
            acc = jnp.zeros((16,), jnp.float32)
            for k in range(D // 16):
                sl = pl.ds(k * 16, 16)
                acc = acc + su[j, sl] * sv[j, sl]
                acc = acc - ru[j, sl] * rv[j, sl]
            dots_v[j, :] = acc
            return 0
        lax.fori_loop(0, PC, pair, 0, unroll=2)

        lanes = lax.iota(jnp.int32, 16)
        for gg in range(PC // 16):
            sl = pl.ds(c * PC + gg * 16, 16)
            rows = lanes + gg * 16
            tot = jnp.full((16,), 1.0, jnp.float32)
            for l in range(16):
                tot = tot + plsc.load_gather(
                    dots_v, [rows, jnp.full((16,), l, jnp.int32)])
            bterm = bv[2][sl] + bv[3][sl] - bv[0][sl] - bv[1][sl]
            out_v[pl.ds(gg * 16, 16)] = jnp.maximum(tot + bterm, 0.0)
        pltpu.sync_copy(out_v, out_hbm.at[pl.ds(pbase + c * PC, PC)])


def _score(h_item, pu, pv, nu, nv, bias, npad):
    pp = pu.shape[0]
    n_chunks = pp // (NW * PC)
    ppw = n_chunks * PC
    scratch = [pltpu.VMEM((ppw,), jnp.int32)] * 4
    scratch += [pltpu.VMEM((ppw,), jnp.float32)] * 4
    scratch += [pltpu.VMEM((PC, D), jnp.float32)] * 8
    scratch += [pltpu.VMEM((PC, 16), jnp.float32),
                pltpu.VMEM((PC,), jnp.float32)]
    scratch += [pltpu.SemaphoreType.DMA] * 3
    body = functools.partial(_score_body, n_chunks)
    fn = pl.kernel(body, out_type=jax.ShapeDtypeStruct((pp,), jnp.float32),
                   mesh=_MESH, scratch_types=tuple(scratch),
                   compiler_params=pltpu.CompilerParams(
                       needs_layout_passes=False))
    return fn(h_item, pu, pv, nu, nv, bias)


def _round_up(a, b):
    return (a + b - 1) // b * b


def kernel(x, edge_index, edge_weights, pos_edge_index, neg_edge_index,
           W_proj, b_proj, Q0, bQ0, W0, bW0, Q1, bQ1, W1, bW1, scorer_bias):
    n = x.shape[0]
    e = edge_index.shape[1]
    p = pos_edge_index.shape[1]

    npad = _round_up(n, NS * EC)
    B = 512
    npad = _round_up(npad, B)

    ep = _round_up(e, NW * EC * NBUF)
    ppad = _round_up(p, NW * PC)

    xp = jnp.pad(x, ((0, npad - n), (0, 0)))
    biasp = jnp.pad(scorer_bias, (0, npad - n))

    fill = (jnp.arange(ep - e, dtype=jnp.int32) % n).astype(jnp.int32)
    src = jnp.concatenate([edge_index[0], fill]).reshape(ep // EC, 1, EC)
    dst = jnp.concatenate([edge_index[1], fill]).reshape(ep // EC, 1, EC)
    w2 = jnp.concatenate([edge_weights,
                          jnp.zeros((ep - e,), jnp.float32)]
                         ).reshape(ep // EC, EC)
    comb = jnp.concatenate([src, dst], axis=1)

    pfill = jnp.zeros((ppad - p,), jnp.int32)
    pu = jnp.concatenate([pos_edge_index[0], pfill])
    pv = jnp.concatenate([pos_edge_index[1], pfill])
    nu = jnp.concatenate([neg_edge_index[0], pfill])
    nv = jnp.concatenate([neg_edge_index[1], pfill])

    h, n0 = _pre(xp, W_proj, b_proj, Q0, bQ0, npad, B)
    aggp0, wsp = _segment_sum(n0, comb, w2, npad, with_ws=True)
    h1, n1 = _combine0(aggp0, wsp, h, W0[:D], W0[D:], bW0, Q1, bQ1, npad, B)
    (aggp1,) = _segment_sum(n1, comb, w2, npad, with_ws=False)
    h_item = _combine1(aggp1, wsp, h1, h, W1[:D], W1[D:], bW1, npad, B)
    scores = _score(h_item, pu, pv, nu, nv, biasp, npad)
    return scores[:p]


# speedup vs baseline: 2.0933x; 1.2961x over previous
import functools

import jax
import jax.numpy as jnp
import numpy as np
from jax import lax
from jax.experimental import pallas as pl
from jax.experimental.pallas import tpu as pltpu
from jax.experimental.pallas import tpu_sc as plsc

D = 128
NC = 2
NS = 16
NW = NC * NS
EC = 80
PC = 80

_MESH = plsc.VectorSubcoreMesh(
    core_axis_name="c", subcore_axis_name="s", num_cores=NC, num_subcores=NS)


def _pre_body(x_ref, wp_ref, bp_ref, q0_ref, bq0_ref, h_ref, n0_ref):
    h = jnp.dot(x_ref[...], wp_ref[...], preferred_element_type=jnp.float32)
    h = h + bp_ref[...]
    h_ref[...] = h
    n0 = jnp.dot(h, q0_ref[...], preferred_element_type=jnp.float32)
    n0_ref[...] = jnp.maximum(n0 + bq0_ref[...], 0.0)


def _pre(x, W_proj, b_proj, Q0, bQ0, NP, B):
    grid = NP // B
    row = pl.BlockSpec((B, D), lambda i: (i, 0))
    full = pl.BlockSpec((D, D), lambda i: (0, 0))
    vec = pl.BlockSpec((1, D), lambda i: (0, 0))
    return pl.pallas_call(
        _pre_body,
        grid=(grid,),
        in_specs=[row, full, vec, full, vec],
        out_specs=[row, row],
        out_shape=[jax.ShapeDtypeStruct((NP, D), jnp.float32)] * 2,
    )(x, W_proj, b_proj.reshape(1, D), Q0, bQ0.reshape(1, D))


def _norm_rows(z):
    nrm = jnp.sqrt(jnp.sum(z * z, axis=1, keepdims=True))
    nrm = jnp.where(nrm == 0.0, 1.0, nrm)
    return z / nrm


def _combine0_body(aggp_ref, wsp_ref, h_ref, wt_ref, wb_ref, bw_ref,
                   q1_ref, bq1_ref, h1_ref, n1_ref):
    agg = aggp_ref[0] + aggp_ref[1]
    ws = jnp.maximum(wsp_ref[0] + wsp_ref[1], 1.0)
    a = agg / ws[:, None]
    z = (jnp.dot(a, wt_ref[...], preferred_element_type=jnp.float32)
         + jnp.dot(h_ref[...], wb_ref[...], preferred_element_type=jnp.float32)
         + bw_ref[...])
    h1 = _norm_rows(jnp.maximum(z, 0.0))
    h1_ref[...] = h1
    n1 = jnp.dot(h1, q1_ref[...], preferred_element_type=jnp.float32)
    n1_ref[...] = jnp.maximum(n1 + bq1_ref[...], 0.0)


def _combine0(aggp, wsp, h, Wt, Wb, bW, Q1, bQ1, NP, B):
    grid = NP // B
    row = pl.BlockSpec((B, D), lambda i: (i, 0))
    full = pl.BlockSpec((D, D), lambda i: (0, 0))
    vec = pl.BlockSpec((1, D), lambda i: (0, 0))
    a3 = pl.BlockSpec((NC, B, D), lambda i: (0, i, 0))
    a2 = pl.BlockSpec((NC, B), lambda i: (0, i))
    return pl.pallas_call(
        _combine0_body,
        grid=(grid,),
        in_specs=[a3, a2, row, full, full, vec, full, vec],
        out_specs=[row, row],
        out_shape=[jax.ShapeDtypeStruct((NP, D), jnp.float32)] * 2,
    )(aggp, wsp, h, Wt, Wb, bW.reshape(1, D), Q1, bQ1.reshape(1, D))


def _combine1_body(aggp_ref, wsp_ref, h1_ref, h_ref, wt_ref, wb_ref, bw_ref,
                   out_ref):
    agg = aggp_ref[0] + aggp_ref[1]
    ws = jnp.maximum(wsp_ref[0] + wsp_ref[1], 1.0)
    a = agg / ws[:, None]
    z = (jnp.dot(a, wt_ref[...], preferred_element_type=jnp.float32)
         + jnp.dot(h1_ref[...], wb_ref[...], preferred_element_type=jnp.float32)
         + bw_ref[...])
    out_ref[...] = h_ref[...] + _norm_rows(jnp.maximum(z, 0.0))


def _combine1(aggp, wsp, h1, h, Wt, Wb, bW, NP, B):
    grid = NP // B
    row = pl.BlockSpec((B, D), lambda i: (i, 0))
    full = pl.BlockSpec((D, D), lambda i: (0, 0))
    vec = pl.BlockSpec((1, D), lambda i: (0, 0))
    a3 = pl.BlockSpec((NC, B, D), lambda i: (0, i, 0))
    a2 = pl.BlockSpec((NC, B), lambda i: (0, i))
    return pl.pallas_call(
        _combine1_body,
        grid=(grid,),
        in_specs=[a3, a2, row, row, full, full, vec],
        out_specs=row,
        out_shape=jax.ShapeDtypeStruct((NP, D), jnp.float32),
    )(aggp, wsp, h1, h, Wt, Wb, bW.reshape(1, D))


NBUF = 4
NIB = 3


def _scale_chunk(rows_b, wgrp, pr, b):
    prv = jnp.full((16,), pr, jnp.int32)
    bv = jnp.full((16,), b, jnp.int32)

    def edge(j, _):
        wvec = plsc.load_gather(wgrp, [prv, bv, jnp.full((16,), j, jnp.int32)])
        for k in range(D // 16):
            sl = pl.ds(k * 16, 16)
            rows_b[j, sl] = rows_b[j, sl] * wvec
        return 0
    lax.fori_loop(0, EC, edge, 0, unroll=2)


def _seg_body(with_ws, n_chunks, npad, n_hbm, comb_hbm, w_hbm, *rest):
    rest = list(rest)
    agg_out = rest.pop(0)
    ws_out = rest.pop(0) if with_ws else None
    comb = rest.pop(0)
    wgrp = rest.pop(0)
    rows = [rest.pop(0) for _ in range(NBUF)]
    zws_v = rest.pop(0) if with_ws else None
    agg_sh = rest.pop(0)
    ws_sh = rest.pop(0) if with_ws else None
    g = [rest.pop(0) for _ in range(NBUF)]
    s = [rest.pop(0) for _ in range(NBUF)]
    isem = rest.pop(0)
    wsem = rest.pop(0) if with_ws else None

    cid = lax.axis_index("c")
    sid = lax.axis_index("s")
    wid = sid * NC + cid
    rpt = npad // NS
    base = sid * rpt
    cbase = wid * n_chunks
    groups = n_chunks // NBUF

    def zrow(i, _):
        for k in range(D // 16):
            rows[0][i, pl.ds(k * 16, 16)] = jnp.zeros((16,), jnp.float32)
        return 0
    lax.fori_loop(0, EC, zrow, 0)
    for r in range(rpt // EC):
        pltpu.sync_copy(rows[0], agg_sh.at[pl.ds(base + r * EC, EC)])
    if with_ws:
        def zws(i, _):
            zws_v[pl.ds(i * 16, 16)] = jnp.zeros((16,), jnp.float32)
            return 0
        lax.fori_loop(0, rpt // 16, zws, 0)
        pltpu.sync_copy(zws_v, ws_sh.at[pl.ds(base, rpt)])

    def _iload(grp, buf, sem_start):
        hslice = comb_hbm.at[pl.ds(cbase + grp * NBUF, NBUF)]
        wslice = w_hbm.at[pl.ds(cbase + grp * NBUF, NBUF)]
        if sem_start:
            pltpu.async_copy(hslice, comb.at[buf], isem)
            pltpu.async_copy(wslice, wgrp.at[buf], isem)
        else:
            pltpu.make_async_copy(hslice, comb.at[buf], isem).wait()
            pltpu.make_async_copy(wslice, wgrp.at[buf], isem).wait()

    pltpu.sync_copy(comb_hbm.at[pl.ds(cbase, NBUF)], comb.at[0])
    pltpu.sync_copy(w_hbm.at[pl.ds(cbase, NBUF)], wgrp.at[0])
    _iload(1, 1, True)
    plsc.subcore_barrier()

    pltpu.async_copy(n_hbm.at[comb.at[0, 0, 0]], rows[0], g[0])
    pltpu.async_copy(n_hbm.at[comb.at[0, 1, 0]], rows[1], g[1])

    def group(p, _):
        pr = p % NIB
        prn = (p + 1) % NIB
        prp = (p - 1) % NIB
        pl.when(p < groups - 1)(lambda: _iload(p + 1, prn, False))
        for b in range(NBUF):
            i = p * NBUF + b
            if b == 2:
                pl.when(p < groups - 2)(
                    lambda: _iload(p + 2, (p + 2) % NIB, True))
            pltpu.make_async_copy(n_hbm.at[comb.at[pr, b, 0]], rows[b],
                                  g[b]).wait()
            bm2 = (b + 2) % NBUF
            spr = prp if b < 2 else pr
            gpr = pr if b < 2 else prn
            gb = b + 2 if b < 2 else b - 2

            def _swait():
                pltpu.make_async_copy(rows[bm2],
                                      agg_sh.at[comb.at[spr, bm2, 1]],
                                      s[bm2]).wait()

            def _gstart():
                pltpu.async_copy(n_hbm.at[comb.at[gpr, gb, 0]], rows[bm2],
                                 g[bm2])

            if b < 2:
                pl.when(p >= 1)(_swait)
                _gstart()
            else:
                _swait()
                pl.when(p < groups - 1)(_gstart)
            _scale_chunk(rows[b], wgrp, pr, b)
            if with_ws:
                bm1 = (b - 1) % NBUF
                wpr = prp if b == 0 else pr

                def _wswait():
                    pltpu.make_async_copy(
                        wgrp.at[wpr, bm1], ws_sh.at[comb.at[wpr, bm1, 1]],
                        wsem).wait()
                if b == 0:
                    pl.when(p >= 1)(_wswait)
                else:
                    _wswait()
                pltpu.async_copy(wgrp.at[pr, b], ws_sh.at[comb.at[pr, b, 1]],
                                 wsem, add=True)
            pltpu.async_copy(rows[b], agg_sh.at[comb.at[pr, b, 1]], s[b],
                             add=True)
        return 0
    lax.fori_loop(0, groups, group, 0)

    lpr = (groups - 1) % NIB
    pltpu.make_async_copy(rows[2], agg_sh.at[comb.at[lpr, 2, 1]],
                          s[2]).wait()
    pltpu.make_async_copy(rows[3], agg_sh.at[comb.at[lpr, 3, 1]],
                          s[3]).wait()
    if with_ws:
        pltpu.make_async_copy(wgrp.at[lpr, NBUF - 1],
                              ws_sh.at[comb.at[lpr, NBUF - 1, 1]],
                              wsem).wait()
    plsc.subcore_barrier()

    pltpu.sync_copy(agg_sh.at[pl.ds(base, rpt)],
                    agg_out.at[cid, pl.ds(base, rpt)])
    if with_ws:
        pltpu.sync_copy(ws_sh.at[pl.ds(base, rpt)],
                        ws_out.at[cid, pl.ds(base, rpt)])


def _segment_sum(n_tab, comb, w2, npad, with_ws):
    n_chunks = comb.shape[0] // NW
    out_type = [jax.ShapeDtypeStruct((NC, npad, D), jnp.float32)]
    scratch = [pltpu.VMEM((NIB, NBUF, 2, EC), jnp.int32),
               pltpu.VMEM((NIB, NBUF, EC), jnp.float32)]
    scratch += [pltpu.VMEM((EC, D), jnp.float32)] * NBUF
    if with_ws:
        out_type.append(jax.ShapeDtypeStruct((NC, npad), jnp.float32))
        scratch.append(pltpu.VMEM((npad // NS,), jnp.float32))
    scratch.append(pltpu.VMEM_SHARED((npad, D), jnp.float32))
    if with_ws:
        scratch.append(pltpu.VMEM_SHARED((npad,), jnp.float32))
    scratch += [pltpu.SemaphoreType.DMA] * (2 * NBUF + 1)
    if with_ws:
        scratch.append(pltpu.SemaphoreType.DMA)

    body = functools.partial(_seg_body, with_ws, n_chunks, npad)
    fn = pl.kernel(body, out_type=tuple(out_type), mesh=_MESH,
                   scratch_types=tuple(scratch),
                   compiler_params=pltpu.CompilerParams(
                       needs_layout_passes=False))
    return fn(n_tab, comb, w2)


def _score_body(n_chunks, h_hbm, pu_hbm, pv_hbm, nu_hbm, nv_hbm, bias_hbm,
                out_hbm, *rest):
    rest = list(rest)
    idx = [rest.pop(0) for _ in range(4)]
    bv = [rest.pop(0) for _ in range(4)]
    rbufs = [[rest.pop(0) for _ in range(4)] for _ in range(2)]
    dots_v = rest.pop(0)
    out_v = rest.pop(0)
    g = [rest.pop(0) for _ in range(2)]
    bsem = rest.pop(0)

    cid = lax.axis_index("c")
    sid = lax.axis_index("s")
    wid = sid * NC + cid
    ppw = n_chunks * PC
    pbase = wid * ppw

    ih = [pu_hbm, pv_hbm, nu_hbm, nv_hbm]
    for t in range(4):
        pltpu.sync_copy(ih[t].at[pl.ds(pbase, ppw)], idx[t])

    def _rgather(c, q, sem_start):
        for t in range(4):
            d = pltpu.make_async_copy(
                h_hbm.at[idx[t].at[pl.ds(c * PC, PC)]], rbufs[q][t], g[q])
            if sem_start:
                d.start()
            else:
                d.wait()

    def _bgather(sem_start):
        for t in range(4):
            for c in range(n_chunks):
                d = pltpu.make_async_copy(
                    bias_hbm.at[idx[t].at[pl.ds(c * PC, PC)]],
                    bv[t].at[pl.ds(c * PC, PC)], bsem)
                if sem_start:
                    d.start()
                else:
                    d.wait()
    _bgather(True)
    _rgather(0, 0, True)
    _bgather(False)

    for c in range(n_chunks):
        q = c % 2
        _rgather(c, q, False)
        if c + 1 < n_chunks:
            _rgather(c + 1, 1 - q, True)
        ru, rv, su, sv = rbufs[q]

        def pair(j, _):
            acc = jnp.zeros((16,), jnp.float32)
            for k in range(D // 16):
                sl = pl.ds(k * 16, 16)
                acc = acc + su[j, sl] * sv[j, sl]
                acc = acc - ru[j, sl] * rv[j, sl]
            dots_v[j, :] = acc
            return 0
        lax.fori_loop(0, PC, pair, 0, unroll=2)

        lanes = lax.iota(jnp.int32, 16)
        for gg in range(PC // 16):
            sl = pl.ds(c * PC + gg * 16, 16)
            rows = lanes + gg * 16
            tot = jnp.full((16,), 1.0, jnp.float32)
            for l in range(16):
                tot = tot + plsc.load_gather(
                    dots_v, [rows, jnp.full((16,), l, jnp.int32)])
            bterm = bv[2][sl] + bv[3][sl] - bv[0][sl] - bv[1][sl]
            out_v[pl.ds(gg * 16, 16)] = jnp.maximum(tot + bterm, 0.0)
        pltpu.sync_copy(out_v, out_hbm.at[pl.ds(pbase + c * PC, PC)])


def _score(h_item, pu, pv, nu, nv, bias, npad):
    pp = pu.shape[0]
    n_chunks = pp // (NW * PC)
    ppw = n_chunks * PC
    scratch = [pltpu.VMEM((ppw,), jnp.int32)] * 4
    scratch += [pltpu.VMEM((ppw,), jnp.float32)] * 4
    scratch += [pltpu.VMEM((PC, D), jnp.float32)] * 8
    scratch += [pltpu.VMEM((PC, 16), jnp.float32),
                pltpu.VMEM((PC,), jnp.float32)]
    scratch += [pltpu.SemaphoreType.DMA] * 3
    body = functools.partial(_score_body, n_chunks)
    fn = pl.kernel(body, out_type=jax.ShapeDtypeStruct((pp,), jnp.float32),
                   mesh=_MESH, scratch_types=tuple(scratch),
                   compiler_params=pltpu.CompilerParams(
                       needs_layout_passes=False))
    return fn(h_item, pu, pv, nu, nv, bias)


def _round_up(a, b):
    return (a + b - 1) // b * b


def kernel(x, edge_index, edge_weights, pos_edge_index, neg_edge_index,
           W_proj, b_proj, Q0, bQ0, W0, bW0, Q1, bQ1, W1, bW1, scorer_bias):
    n = x.shape[0]
    e = edge_index.shape[1]
    p = pos_edge_index.shape[1]

    npad = _round_up(n, NS * EC)
    B = 512
    npad = _round_up(npad, B)

    ep = _round_up(e, NW * EC * NBUF)
    ppad = _round_up(p, NW * PC)

    xp = jnp.pad(x, ((0, npad - n), (0, 0)))
    biasp = jnp.pad(scorer_bias, (0, npad - n))

    fill = (jnp.arange(ep - e, dtype=jnp.int32) % n).astype(jnp.int32)
    src = jnp.concatenate([edge_index[0], fill]).reshape(ep // EC, 1, EC)
    dst = jnp.concatenate([edge_index[1], fill]).reshape(ep // EC, 1, EC)
    w2 = jnp.concatenate([edge_weights,
                          jnp.zeros((ep - e,), jnp.float32)]
                         ).reshape(ep // EC, EC)
    comb = jnp.concatenate([src, dst], axis=1)

    pfill = (jnp.arange(ppad - p, dtype=jnp.int32) % n).astype(jnp.int32)
    pu = jnp.concatenate([pos_edge_index[0], pfill])
    pv = jnp.concatenate([pos_edge_index[1], pfill])
    nu = jnp.concatenate([neg_edge_index[0], pfill])
    nv = jnp.concatenate([neg_edge_index[1], pfill])

    h, n0 = _pre(xp, W_proj, b_proj, Q0, bQ0, npad, B)
    aggp0, wsp = _segment_sum(n0, comb, w2, npad, with_ws=True)
    h1, n1 = _combine0(aggp0, wsp, h, W0[:D], W0[D:], bW0, Q1, bQ1, npad, B)
    (aggp1,) = _segment_sum(n1, comb, w2, npad, with_ws=False)
    h_item = _combine1(aggp1, wsp, h1, h, W1[:D], W1[D:], bW1, npad, B)
    scores = _score(h_item, pu, pv, nu, nv, biasp, npad)
    return scores[:p]
